# Initial kernel scaffold; baseline (speedup 1.0000x reference)
import jax
import jax.numpy as jnp
from jax.experimental import pallas as pl


def kernel(data, row_matrix, col_matrix, num_vertices, Wr, br, Wc, bc):
    raise NotImplementedError("write your pallas kernel here")


# speedup vs baseline: 6115.1616x; 6115.1616x over previous
import functools

import jax
import jax.numpy as jnp
from jax.experimental import pallas as pl
from jax.experimental.pallas import tpu as pltpu


def _body(feat_ref, featT_ref, mr_ref, mc_ref, wr_ref, wrT_ref, br_ref,
          wc_ref, wcT_ref, bc_ref, out_ref, acc_ref):
    step = pl.program_id(0)
    nsteps = pl.num_programs(0)

    @pl.when(step == 0)
    def _init():
        acc_ref[0] = 0.0
        acc_ref[1] = 0.0
        acc_ref[2] = 0.0
        acc_ref[3] = 0.0

    feat_blk = feat_ref[...]
    featT = featT_ref[...]

    def one_matrix(m_ref, w_ref, wT_ref, b_ref, slot):
        m = m_ref[...]
        w = w_ref[...]
        wT = wT_ref[...]
        dw_col = w[:, 1:2] - w[:, 0:1]
        dw_row = wT[1:2, :] - wT[0:1, :]
        db = b_ref[1] - b_ref[0]
        u_rows = jnp.sum(feat_blk * dw_row, axis=1, keepdims=True)
        u_cols = jnp.sum(featT * dw_col, axis=0, keepdims=True)
        d = u_rows - u_cols + db
        t = (m != 0).astype(jnp.float32)
        sp = jnp.maximum(d, 0.0) + jnp.log(1.0 + jnp.exp(-jnp.abs(d)))
        wnll = (0.2 + 0.8 * t) * sp - t * d
        acc_ref[slot] += jnp.sum(wnll)
        acc_ref[slot + 1] += jnp.sum(t)

    one_matrix(mr_ref, wr_ref, wrT_ref, br_ref, 0)
    one_matrix(mc_ref, wc_ref, wcT_ref, bc_ref, 2)

    @pl.when(step == nsteps - 1)
    def _fin():
        v = featT_ref.shape[1]
        total = float(v) * float(v)
        loss_r = acc_ref[0] / (0.2 * total + 0.8 * acc_ref[1])
        loss_c = acc_ref[2] / (0.2 * total + 0.8 * acc_ref[3])
        out_ref[0] = loss_r + loss_c


def kernel(data, row_matrix, col_matrix, num_vertices, Wr, br, Wc, bc):
    del num_vertices
    V = row_matrix.shape[1]
    feat = data[0, :, :4]
    featT = jnp.transpose(feat)
    mr = row_matrix[0]
    mc = col_matrix[0]
    BR = 256
    nsteps = V // BR

    out = pl.pallas_call(
        _body,
        grid=(nsteps,),
        in_specs=[
            pl.BlockSpec((BR, 4), lambda i: (i, 0)),
            pl.BlockSpec((4, V), lambda i: (0, 0)),
            pl.BlockSpec((BR, V), lambda i: (i, 0)),
            pl.BlockSpec((BR, V), lambda i: (i, 0)),
            pl.BlockSpec((4, 2), lambda i: (0, 0)),
            pl.BlockSpec((2, 4), lambda i: (0, 0)),
            pl.BlockSpec(memory_space=pltpu.SMEM),
            pl.BlockSpec((4, 2), lambda i: (0, 0)),
            pl.BlockSpec((2, 4), lambda i: (0, 0)),
            pl.BlockSpec(memory_space=pltpu.SMEM),
        ],
        out_specs=pl.BlockSpec(memory_space=pltpu.SMEM),
        out_shape=jax.ShapeDtypeStruct((1,), jnp.float32),
        scratch_shapes=[pltpu.SMEM((4,), jnp.float32)],
        compiler_params=pltpu.CompilerParams(
            dimension_semantics=("arbitrary",),
        ),
    )(feat, featT, mr, mc, Wr, jnp.transpose(Wr), br, Wc, jnp.transpose(Wc), bc)
    return out
